# Initial kernel scaffold; baseline (speedup 1.0000x reference)
import jax
import jax.numpy as jnp
from jax.experimental import pallas as pl


def kernel(atom_types, embedding_weight):
    raise NotImplementedError("write your pallas kernel here")


# speedup vs baseline: 5.0187x; 5.0187x over previous
import functools

import jax
import jax.numpy as jnp
from jax import lax
from jax.experimental import pallas as pl
from jax.experimental.pallas import tpu as pltpu
from jax.experimental.pallas import tpu_sc as plsc

_GATHER_DNUMS = lax.GatherDimensionNumbers(
    offset_dims=(), collapsed_slice_dims=(0,), start_index_map=(0,))


def _bcast_lane(v16, n):
    return lax.gather(v16, jnp.full((16, 1), n, jnp.int32), _GATHER_DNUMS,
                      slice_sizes=(1,),
                      mode=lax.GatherScatterMode.PROMISE_IN_BOUNDS)


_N = 16384 * 200
_D = 64
_NW = 32
_RPW = _N // _NW
_C = 512
_STEPS = _RPW // _C


def _sc_embed(idx_hbm, w_hbm, out_hbm, idx_v, out_v, w_v):
    wid = lax.axis_index("s") * 2 + lax.axis_index("c")
    base = wid * _RPW
    pltpu.sync_copy(w_hbm, w_v)
    w0 = [w_v[pl.ds(q * 16, 16)] for q in range(4)]
    dw = [w_v[pl.ds(_D + q * 16, 16)] - w0[q] for q in range(4)]

    def step(i, carry):
        row0 = base + i * _C
        pltpu.sync_copy(idx_hbm.at[pl.ds(row0, _C)], idx_v)

        def group(g, c2):
            t16 = idx_v[pl.ds(g * 16, 16)]
            gbase = g * (16 * _D)
            for n in range(16):
                tf = _bcast_lane(t16, n).astype(jnp.float32)
                for q in range(4):
                    out_v[pl.ds(gbase + n * _D + q * 16, 16)] = (
                        w0[q] + tf * dw[q])
            return c2

        lax.fori_loop(0, _C // 16, group, 0)
        pltpu.sync_copy(out_v, out_hbm.at[pl.ds(row0 * _D, _C * _D)])
        return carry

    lax.fori_loop(0, _STEPS, step, 0)


def kernel(atom_types, embedding_weight):
    shape = atom_types.shape
    idx = atom_types.reshape(-1).astype(jnp.int32)
    wflat = embedding_weight.reshape(-1).astype(jnp.float32)
    mesh = plsc.VectorSubcoreMesh(core_axis_name="c", subcore_axis_name="s")
    run = functools.partial(
        pl.kernel,
        mesh=mesh,
        out_type=jax.ShapeDtypeStruct((_N * _D,), jnp.float32),
        scratch_types=[
            pltpu.VMEM((_C,), jnp.int32),
            pltpu.VMEM((_C * _D,), jnp.float32),
            pltpu.VMEM((2 * _D,), jnp.float32),
        ],
    )(_sc_embed)
    out = run(idx, wflat)
    return out.reshape(*shape, _D)


# speedup vs baseline: 5.8996x; 1.1755x over previous
import functools

import jax
import jax.numpy as jnp
from jax import lax
from jax.experimental import pallas as pl
from jax.experimental.pallas import tpu as pltpu
from jax.experimental.pallas import tpu_sc as plsc

_GATHER_DNUMS = lax.GatherDimensionNumbers(
    offset_dims=(), collapsed_slice_dims=(0,), start_index_map=(0,))


def _bcast_lane(v16, n):
    return lax.gather(v16, jnp.full((16, 1), n, jnp.int32), _GATHER_DNUMS,
                      slice_sizes=(1,),
                      mode=lax.GatherScatterMode.PROMISE_IN_BOUNDS)


_N = 16384 * 200
_D = 64
_NW = 32
_RPW = _N // _NW
_C = 800
_STEPS = _RPW // _C


def _sc_embed(idx_hbm, w_hbm, out_hbm,
              idx_v0, idx_v1, out_v0, out_v1, w_v,
              sem_i0, sem_i1, sem_o0, sem_o1):
    wid = lax.axis_index("s") * 2 + lax.axis_index("c")
    base = wid * _RPW
    idx_bufs = (idx_v0, idx_v1)
    out_bufs = (out_v0, out_v1)
    sem_i = (sem_i0, sem_i1)
    sem_o = (sem_o0, sem_o1)

    pltpu.sync_copy(w_hbm, w_v)
    w0 = [w_v[pl.ds(q * 16, 16)] for q in range(4)]
    dw = [w_v[pl.ds(_D + q * 16, 16)] - w0[q] for q in range(4)]

    def start_in(s, b):
        pltpu.make_async_copy(
            idx_hbm.at[pl.ds(base + s * _C, _C)], idx_bufs[b], sem_i[b]
        ).start()

    def wait_in(s, b):
        pltpu.make_async_copy(
            idx_hbm.at[pl.ds(base + s * _C, _C)], idx_bufs[b], sem_i[b]
        ).wait()

    def start_out(s, b):
        pltpu.make_async_copy(
            out_bufs[b], out_hbm.at[pl.ds((base + s * _C) * _D, _C * _D)],
            sem_o[b]).start()

    def wait_out(s, b):
        pltpu.make_async_copy(
            out_bufs[b], out_hbm.at[pl.ds((base + s * _C) * _D, _C * _D)],
            sem_o[b]).wait()

    def compute(b):
        out_v = out_bufs[b]
        idx_v = idx_bufs[b]

        def group(g, c2):
            t16 = idx_v[pl.ds(g * 16, 16)]
            gbase = g * (16 * _D)
            for n in range(16):
                tf = _bcast_lane(t16, n).astype(jnp.float32)
                for q in range(4):
                    out_v[pl.ds(gbase + n * _D + q * 16, 16)] = (
                        w0[q] + tf * dw[q])
            return c2

        lax.fori_loop(0, _C // 16, group, 0)

    start_in(0, 0)
    start_in(1, 1)
    for b in range(2):
        wait_in(b, b)
        compute(b)
        start_out(b, b)
        start_in(b + 2, b)

    def steady(j, carry):
        s = 2 * j
        for b in range(2):
            wait_in(s + b, b)
            wait_out(s + b - 2, b)
            compute(b)
            start_out(s + b, b)

            @pl.when(s + b + 2 < _STEPS)
            def _():
                start_in(s + b + 2, b)
        return carry

    lax.fori_loop(1, _STEPS // 2, steady, 0)
    wait_out(_STEPS - 2, 0)
    wait_out(_STEPS - 1, 1)


def kernel(atom_types, embedding_weight):
    shape = atom_types.shape
    idx = atom_types.reshape(-1).astype(jnp.int32)
    wflat = embedding_weight.reshape(-1).astype(jnp.float32)
    mesh = plsc.VectorSubcoreMesh(core_axis_name="c", subcore_axis_name="s")
    run = functools.partial(
        pl.kernel,
        mesh=mesh,
        out_type=jax.ShapeDtypeStruct((_N * _D,), jnp.float32),
        scratch_types=[
            pltpu.VMEM((_C,), jnp.int32),
            pltpu.VMEM((_C,), jnp.int32),
            pltpu.VMEM((_C * _D,), jnp.float32),
            pltpu.VMEM((_C * _D,), jnp.float32),
            pltpu.VMEM((2 * _D,), jnp.float32),
            pltpu.SemaphoreType.DMA,
            pltpu.SemaphoreType.DMA,
            pltpu.SemaphoreType.DMA,
            pltpu.SemaphoreType.DMA,
        ],
    )(_sc_embed)
    out = run(idx, wflat)
    return out.reshape(*shape, _D)


# speedup vs baseline: 5.9077x; 1.0014x over previous
import functools

import jax
import jax.numpy as jnp
from jax import lax
from jax.experimental import pallas as pl
from jax.experimental.pallas import tpu as pltpu
from jax.experimental.pallas import tpu_sc as plsc

_GATHER_DNUMS = lax.GatherDimensionNumbers(
    offset_dims=(), collapsed_slice_dims=(0,), start_index_map=(0,))


def _bcast_lane(v16, n):
    return lax.gather(v16, jnp.full((16, 1), n, jnp.int32), _GATHER_DNUMS,
                      slice_sizes=(1,),
                      mode=lax.GatherScatterMode.PROMISE_IN_BOUNDS)


_N = 16384 * 200
_D = 64
_NW = 32
_RPW = _N // _NW
_C = 800
_STEPS = _RPW // _C


def _sc_embed(idx_hbm, w_hbm, out_hbm,
              idx_v0, idx_v1, out_v0, out_v1, w_v,
              sem_i0, sem_i1, sem_o0, sem_o1):
    wid = lax.axis_index("s") * 2 + lax.axis_index("c")
    base = wid * _RPW
    idx_bufs = (idx_v0, idx_v1)
    out_bufs = (out_v0, out_v1)
    sem_i = (sem_i0, sem_i1)
    sem_o = (sem_o0, sem_o1)

    pltpu.sync_copy(w_hbm, w_v)
    w0 = [w_v[pl.ds(q * 16, 16)] for q in range(4)]
    dw = [w_v[pl.ds(_D + q * 16, 16)] - w0[q] for q in range(4)]

    def start_in(s, b):
        pltpu.make_async_copy(
            idx_hbm.at[pl.ds(base + s * _C, _C)], idx_bufs[b], sem_i[b]
        ).start()

    def wait_in(s, b):
        pltpu.make_async_copy(
            idx_hbm.at[pl.ds(base + s * _C, _C)], idx_bufs[b], sem_i[b]
        ).wait()

    def start_out(s, b):
        pltpu.make_async_copy(
            out_bufs[b], out_hbm.at[pl.ds((base + s * _C) * _D, _C * _D)],
            sem_o[b]).start()

    def wait_out(s, b):
        pltpu.make_async_copy(
            out_bufs[b], out_hbm.at[pl.ds((base + s * _C) * _D, _C * _D)],
            sem_o[b]).wait()

    def compute(b):
        out_v = out_bufs[b]
        idx_v = idx_bufs[b]

        @plsc.parallel_loop(0, _C // 16, unroll=2)
        def group(g):
            tf16 = idx_v[pl.ds(g * 16, 16)].astype(jnp.float32)
            gbase = g * (16 * _D)
            for n in range(16):
                tf = _bcast_lane(tf16, n)
                for q in range(4):
                    out_v[pl.ds(gbase + n * _D + q * 16, 16)] = (
                        w0[q] + tf * dw[q])

    start_in(0, 0)
    start_in(1, 1)
    for b in range(2):
        wait_in(b, b)
        compute(b)
        start_out(b, b)
        start_in(b + 2, b)

    def steady(j, carry):
        s = 2 * j
        for b in range(2):
            wait_in(s + b, b)
            wait_out(s + b - 2, b)
            compute(b)
            start_out(s + b, b)

            @pl.when(s + b + 2 < _STEPS)
            def _():
                start_in(s + b + 2, b)
        return carry

    lax.fori_loop(1, _STEPS // 2, steady, 0)
    wait_out(_STEPS - 2, 0)
    wait_out(_STEPS - 1, 1)


def kernel(atom_types, embedding_weight):
    shape = atom_types.shape
    idx = atom_types.reshape(-1).astype(jnp.int32)
    wflat = embedding_weight.reshape(-1).astype(jnp.float32)
    mesh = plsc.VectorSubcoreMesh(core_axis_name="c", subcore_axis_name="s")
    run = functools.partial(
        pl.kernel,
        mesh=mesh,
        out_type=jax.ShapeDtypeStruct((_N * _D,), jnp.float32),
        scratch_types=[
            pltpu.VMEM((_C,), jnp.int32),
            pltpu.VMEM((_C,), jnp.int32),
            pltpu.VMEM((_C * _D,), jnp.float32),
            pltpu.VMEM((_C * _D,), jnp.float32),
            pltpu.VMEM((2 * _D,), jnp.float32),
            pltpu.SemaphoreType.DMA,
            pltpu.SemaphoreType.DMA,
            pltpu.SemaphoreType.DMA,
            pltpu.SemaphoreType.DMA,
        ],
    )(_sc_embed)
    out = run(idx, wflat)
    return out.reshape(*shape, _D)


# speedup vs baseline: 32.4389x; 5.4910x over previous
import functools

import jax
import jax.numpy as jnp
from jax import lax
from jax.experimental import pallas as pl
from jax.experimental.pallas import tpu as pltpu
from jax.experimental.pallas import tpu_sc as plsc

_GATHER_DNUMS = lax.GatherDimensionNumbers(
    offset_dims=(), collapsed_slice_dims=(0,), start_index_map=(0,))


def _bcast_lane(v16, n):
    return lax.gather(v16, jnp.full((16, 1), n, jnp.int32), _GATHER_DNUMS,
                      slice_sizes=(1,),
                      mode=lax.GatherScatterMode.PROMISE_IN_BOUNDS)


_B = 16384
_J = 200
_D = 64
_NW = 32
_CI = _B // _NW


def _sc_embed(idx_hbm, w_hbm, out_hbm,
              idx_v0, idx_v1, buf0, buf1, w_v, w0s, dws,
              sem_i0, sem_i1, sem_o0, sem_o1):
    wid = lax.axis_index("s") * 2 + lax.axis_index("c")
    i_lo = wid * _CI
    idx_bufs = (idx_v0, idx_v1)
    out_bufs = (buf0, buf1)
    sem_i = (sem_i0, sem_i1)
    sem_o = (sem_o0, sem_o1)

    pltpu.sync_copy(w_hbm, w_v)
    for k in range(_D):
        a = _bcast_lane(w_v[pl.ds((k // 16) * 16, 16)], k % 16)
        b = _bcast_lane(w_v[pl.ds(_D + (k // 16) * 16, 16)], k % 16)
        w0s.at[k][:] = a
        dws.at[k][:] = b - a

    def start_in(j, b):
        pltpu.make_async_copy(
            idx_hbm.at[j, pl.ds(i_lo, _CI)], idx_bufs[b], sem_i[b]).start()

    def wait_in(j, b):
        pltpu.make_async_copy(
            idx_hbm.at[j, pl.ds(i_lo, _CI)], idx_bufs[b], sem_i[b]).wait()

    def start_out(j, b):
        pltpu.make_async_copy(
            out_bufs[b], out_hbm.at[j, :, pl.ds(i_lo, _CI)], sem_o[b]).start()

    def wait_out(j, b):
        pltpu.make_async_copy(
            out_bufs[b], out_hbm.at[j, :, pl.ds(i_lo, _CI)], sem_o[b]).wait()

    def compute(b):
        buf = out_bufs[b]
        idx_v = idx_bufs[b]

        @plsc.parallel_loop(0, _D, unroll=2)
        def row(k):
            w0k = w0s.at[k][:]
            dwk = dws.at[k][:]
            brow = buf.at[k]
            for g in range(_CI // 16):
                tf = idx_v[pl.ds(g * 16, 16)].astype(jnp.float32)
                brow[pl.ds(g * 16, 16)] = w0k + tf * dwk

    start_in(0, 0)
    start_in(1, 1)
    for b in range(2):
        wait_in(b, b)
        compute(b)
        start_out(b, b)
        start_in(b + 2, b)

    def steady(h, carry):
        j = 2 * h
        for b in range(2):
            wait_in(j + b, b)
            wait_out(j + b - 2, b)
            compute(b)
            start_out(j + b, b)

            @pl.when(j + b + 2 < _J)
            def _():
                start_in(j + b + 2, b)
        return carry

    lax.fori_loop(1, _J // 2, steady, 0)
    wait_out(_J - 2, 0)
    wait_out(_J - 1, 1)


def kernel(atom_types, embedding_weight):
    idx_t = atom_types.astype(jnp.int32).T
    wflat = embedding_weight.reshape(-1).astype(jnp.float32)
    mesh = plsc.VectorSubcoreMesh(core_axis_name="c", subcore_axis_name="s")
    run = functools.partial(
        pl.kernel,
        mesh=mesh,
        out_type=jax.ShapeDtypeStruct((_J, _D, _B), jnp.float32),
        compiler_params=pltpu.CompilerParams(use_tc_tiling_on_sc=True),
        scratch_types=[
            pltpu.VMEM((_CI,), jnp.int32),
            pltpu.VMEM((_CI,), jnp.int32),
            pltpu.VMEM((_D, _CI), jnp.float32),
            pltpu.VMEM((_D, _CI), jnp.float32),
            pltpu.VMEM((2 * _D,), jnp.float32),
            pltpu.VMEM((_D, 16), jnp.float32),
            pltpu.VMEM((_D, 16), jnp.float32),
            pltpu.SemaphoreType.DMA,
            pltpu.SemaphoreType.DMA,
            pltpu.SemaphoreType.DMA,
            pltpu.SemaphoreType.DMA,
        ],
    )(_sc_embed)
    out = run(idx_t, wflat)
    return out.transpose(2, 0, 1)


# speedup vs baseline: 41.3709x; 1.2754x over previous
import functools

import jax
import jax.numpy as jnp
from jax import lax
from jax.experimental import pallas as pl
from jax.experimental.pallas import tpu as pltpu
from jax.experimental.pallas import tpu_sc as plsc

_GATHER_DNUMS = lax.GatherDimensionNumbers(
    offset_dims=(), collapsed_slice_dims=(0,), start_index_map=(0,))


def _bcast_lane(v16, n):
    return lax.gather(v16, jnp.full((16, 1), n, jnp.int32), _GATHER_DNUMS,
                      slice_sizes=(1,),
                      mode=lax.GatherScatterMode.PROMISE_IN_BOUNDS)


_B = 16384
_J = 200
_D = 64
_NW = 32
_CI = 4096
_NIC = _B // _CI
_KT = _D // 8
_G = _CI // 16


def _sc_embed(idx_hbm, w_hbm, out_hbm,
              idx_v, tf_v, buf0, buf1, w_v, w0s, dws,
              sem_i, sem_o0, sem_o1):
    wid = lax.axis_index("s") * 2 + lax.axis_index("c")
    m_lo = (wid * _J // _NW) * _NIC
    m_hi = ((wid + 1) * _J // _NW) * _NIC
    out_bufs = (buf0, buf1)
    sem_o = (sem_o0, sem_o1)

    pltpu.sync_copy(w_hbm, w_v)
    for k in range(_D):
        a = _bcast_lane(w_v[pl.ds((k // 16) * 16, 16)], k % 16)
        b = _bcast_lane(w_v[pl.ds(_D + (k // 16) * 16, 16)], k % 16)
        w0s.at[k][:] = a
        dws.at[k][:] = b - a

    def in_copy(m):
        j, ic = m // _NIC, m % _NIC
        return pltpu.make_async_copy(
            idx_hbm.at[j, pl.ds(ic * _CI, _CI)], idx_v, sem_i)

    def out_copy(m, kt, b):
        j, ic = m // _NIC, m % _NIC
        return pltpu.make_async_copy(
            out_bufs[b],
            out_hbm.at[j, pl.ds(kt * 8, 8), pl.ds(ic * _CI, _CI)],
            sem_o[b])

    def build_tf():
        @plsc.parallel_loop(0, _G, unroll=4)
        def cvt(g):
            tf_v[pl.ds(g * 16, 16)] = (
                idx_v[pl.ds(g * 16, 16)].astype(jnp.float32))

    def macro(m, first):
        in_copy(m).wait()
        build_tf()
        in_copy(jnp.minimum(m + 1, m_hi - 1)).start()
        for kt in range(_KT):
            b = kt % 2
            if not (first and kt < 2):
                out_copy(m, kt, b).wait()
            compute_k(m, kt, b)

    def compute_k(m, kt, b):
        buf = out_bufs[b]

        @plsc.parallel_loop(0, 8, unroll=1)
        def row(k):
            w0k = w0s.at[kt * 8 + k][:]
            dwk = dws.at[kt * 8 + k][:]
            brow = buf.at[k]

            @plsc.parallel_loop(0, _G, unroll=8)
            def seg(g):
                tf = tf_v[pl.ds(g * 16, 16)]
                brow[pl.ds(g * 16, 16)] = w0k + tf * dwk

        out_copy(m, kt, b).start()

    in_copy(m_lo).start()
    macro(m_lo, first=True)

    def steady(m, carry):
        macro(m, first=False)
        return carry

    lax.fori_loop(m_lo + 1, m_hi, steady, 0)
    out_copy(m_hi - 1, _KT - 2, 0).wait()
    out_copy(m_hi - 1, _KT - 1, 1).wait()
    in_copy(m_hi - 1).wait()


def kernel(atom_types, embedding_weight):
    idx_t = atom_types.astype(jnp.int32).T
    wflat = embedding_weight.reshape(-1).astype(jnp.float32)
    mesh = plsc.VectorSubcoreMesh(core_axis_name="c", subcore_axis_name="s")
    run = functools.partial(
        pl.kernel,
        mesh=mesh,
        out_type=jax.ShapeDtypeStruct((_J, _D, _B), jnp.float32),
        compiler_params=pltpu.CompilerParams(use_tc_tiling_on_sc=True),
        scratch_types=[
            pltpu.VMEM((_CI,), jnp.int32),
            pltpu.VMEM((_CI,), jnp.float32),
            pltpu.VMEM((8, _CI), jnp.float32),
            pltpu.VMEM((8, _CI), jnp.float32),
            pltpu.VMEM((2 * _D,), jnp.float32),
            pltpu.VMEM((_D, 16), jnp.float32),
            pltpu.VMEM((_D, 16), jnp.float32),
            pltpu.SemaphoreType.DMA,
            pltpu.SemaphoreType.DMA,
            pltpu.SemaphoreType.DMA,
        ],
    )(_sc_embed)
    out = run(idx_t, wflat)
    return out.transpose(2, 0, 1)


# speedup vs baseline: 43.9884x; 1.0633x over previous
import functools

import jax
import jax.numpy as jnp
from jax import lax
from jax.experimental import pallas as pl
from jax.experimental.pallas import tpu as pltpu
from jax.experimental.pallas import tpu_sc as plsc

_GATHER_DNUMS = lax.GatherDimensionNumbers(
    offset_dims=(), collapsed_slice_dims=(0,), start_index_map=(0,))


def _bcast_lane(v16, n):
    return lax.gather(v16, jnp.full((16, 1), n, jnp.int32), _GATHER_DNUMS,
                      slice_sizes=(1,),
                      mode=lax.GatherScatterMode.PROMISE_IN_BOUNDS)


_B = 16384
_J = 200
_D = 64
_NW = 32
_CI = 4096
_NIC = _B // _CI
_KT = _D // 8
_G = _CI // 16


def _sc_embed(idx_hbm, w_hbm, out_hbm,
              idx_v, tf_v, buf0, buf1, w_v, w0s, dws,
              sem_i, sem_o0, sem_o1):
    wid = lax.axis_index("s") * 2 + lax.axis_index("c")
    m_per_w = _J * _NIC // _NW
    m_lo = wid * m_per_w
    m_hi = m_lo + m_per_w
    out_bufs = (buf0, buf1)
    sem_o = (sem_o0, sem_o1)

    pltpu.sync_copy(w_hbm, w_v)
    for k in range(_D):
        a = _bcast_lane(w_v[pl.ds((k // 16) * 16, 16)], k % 16)
        b = _bcast_lane(w_v[pl.ds(_D + (k // 16) * 16, 16)], k % 16)
        w0s.at[k][:] = a
        dws.at[k][:] = b - a

    def in_copy(m):
        j, ic = m // _NIC, m % _NIC
        return pltpu.make_async_copy(
            idx_hbm.at[j, pl.ds(ic * _CI, _CI)], idx_v, sem_i)

    def out_copy(m, kt, b):
        j, ic = m // _NIC, m % _NIC
        return pltpu.make_async_copy(
            out_bufs[b],
            out_hbm.at[j, pl.ds(kt * 8, 8), pl.ds(ic * _CI, _CI)],
            sem_o[b])

    def build_tf():
        @plsc.parallel_loop(0, _G, unroll=4)
        def cvt(g):
            tf_v[pl.ds(g * 16, 16)] = (
                idx_v[pl.ds(g * 16, 16)].astype(jnp.float32))

    def macro(m, first):
        in_copy(m).wait()
        build_tf()
        in_copy(jnp.minimum(m + 1, m_hi - 1)).start()
        for kt in range(_KT):
            b = kt % 2
            if not (first and kt < 2):
                out_copy(m, kt, b).wait()
            compute_k(m, kt, b)

    def compute_k(m, kt, b):
        buf = out_bufs[b]

        @plsc.parallel_loop(0, 8, unroll=1)
        def row(k):
            w0k = w0s.at[kt * 8 + k][:]
            dwk = dws.at[kt * 8 + k][:]
            brow = buf.at[k]

            @plsc.parallel_loop(0, _G, unroll=8)
            def seg(g):
                tf = tf_v[pl.ds(g * 16, 16)]
                brow[pl.ds(g * 16, 16)] = w0k + tf * dwk

        out_copy(m, kt, b).start()

    in_copy(m_lo).start()
    macro(m_lo, first=True)

    def steady(m, carry):
        macro(m, first=False)
        return carry

    lax.fori_loop(m_lo + 1, m_hi, steady, 0)
    out_copy(m_hi - 1, _KT - 2, 0).wait()
    out_copy(m_hi - 1, _KT - 1, 1).wait()
    in_copy(m_hi - 1).wait()


def kernel(atom_types, embedding_weight):
    idx_t = atom_types.astype(jnp.int32).T
    wflat = embedding_weight.reshape(-1).astype(jnp.float32)
    mesh = plsc.VectorSubcoreMesh(core_axis_name="c", subcore_axis_name="s")
    run = functools.partial(
        pl.kernel,
        mesh=mesh,
        out_type=jax.ShapeDtypeStruct((_J, _D, _B), jnp.float32),
        compiler_params=pltpu.CompilerParams(use_tc_tiling_on_sc=True),
        scratch_types=[
            pltpu.VMEM((_CI,), jnp.int32),
            pltpu.VMEM((_CI,), jnp.float32),
            pltpu.VMEM((8, _CI), jnp.float32),
            pltpu.VMEM((8, _CI), jnp.float32),
            pltpu.VMEM((2 * _D,), jnp.float32),
            pltpu.VMEM((_D, 16), jnp.float32),
            pltpu.VMEM((_D, 16), jnp.float32),
            pltpu.SemaphoreType.DMA,
            pltpu.SemaphoreType.DMA,
            pltpu.SemaphoreType.DMA,
        ],
    )(_sc_embed)
    out = run(idx_t, wflat)
    return out.transpose(2, 0, 1)
